# baseline (speedup 1.0000x reference)
import functools

import jax
import jax.numpy as jnp
from jax import lax
from jax.experimental import pallas as pl
from jax.experimental.pallas import tpu as pltpu
from jax.experimental.pallas import tpu_sc as plsc

N = 10000
E = 320000
D_IN = 128
H = 64
G = 128

NC = 2
NS = 16
NW = NC * NS
WIN = 512
NWIN = 20
EPAD = NW * NWIN * WIN - E
ROWS_ACC = 10112
ROWS_D = 10240

ACC_PER_SUB = ROWS_ACC // NS
D_PER_SUB = ROWS_D // NS
STRIPE = 640
TAIL = N - STRIPE * (NS - 1)

_MESH = dict(core_axis_name="c", subcore_axis_name="s",
             num_cores=NC, num_subcores=NS)
_SC_PARAMS = pltpu.CompilerParams(use_tc_tiling_on_sc=False)


def _deg_body(dstp, zeros1, out, dacc, didx, ones):
    c = lax.axis_index("c")
    s = lax.axis_index("s")
    wid = s * NC + c
    pltpu.sync_copy(zeros1.at[pl.ds(s * D_PER_SUB, D_PER_SUB)],
                    dacc.at[pl.ds(s * D_PER_SUB, D_PER_SUB)])
    pltpu.sync_copy(dstp.at[wid], didx)
    for i in range(WIN // 16):
        ones[pl.ds(i * 16, 16)] = jnp.ones((16,), jnp.float32)
    plsc.subcore_barrier()

    def win(w, carry):
        pltpu.sync_copy(ones, dacc.at[didx.at[w]], add=True)
        return carry

    lax.fori_loop(0, NWIN, win, 0, unroll=False)
    plsc.subcore_barrier()
    pltpu.sync_copy(dacc.at[pl.ds(s * D_PER_SUB, D_PER_SUB)],
                    out.at[pl.ds(c * ROWS_D + s * D_PER_SUB, D_PER_SUB)])


def _deg_kernel(dstp, zeros1):
    return pl.kernel(
        _deg_body,
        out_type=jax.ShapeDtypeStruct((NC * ROWS_D,), jnp.float32),
        mesh=plsc.VectorSubcoreMesh(**_MESH),
        compiler_params=_SC_PARAMS,
        scratch_types=[
            pltpu.VMEM_SHARED((ROWS_D,), jnp.float32),
            pltpu.VMEM((NWIN, WIN), jnp.int32),
            pltpu.VMEM((WIN,), jnp.float32),
        ],
    )(dstp, zeros1)


def _agg_body(g_hbm, zeros2, srcp, dstp, out, acc, sidx, didx,
              stg_a, stg_b, sem_a, sem_b):
    c = lax.axis_index("c")
    s = lax.axis_index("s")
    wid = s * NC + c
    pltpu.sync_copy(zeros2.at[pl.ds(s * ACC_PER_SUB, ACC_PER_SUB)],
                    acc.at[pl.ds(s * ACC_PER_SUB, ACC_PER_SUB)])
    pltpu.sync_copy(srcp.at[wid], sidx)
    pltpu.sync_copy(dstp.at[wid], didx)
    plsc.subcore_barrier()

    pltpu.async_copy(g_hbm.at[sidx.at[0]], stg_a, sem_a)

    def pair(k, carry):
        w = 2 * k
        pltpu.make_async_copy(g_hbm.at[sidx.at[w]], stg_a, sem_a).wait()
        pltpu.async_copy(g_hbm.at[sidx.at[w + 1]], stg_b, sem_b)
        pltpu.sync_copy(stg_a, acc.at[didx.at[w]], add=True)
        pltpu.make_async_copy(g_hbm.at[sidx.at[w + 1]], stg_b, sem_b).wait()

        @pl.when(k < NWIN // 2 - 1)
        def _():
            pltpu.async_copy(g_hbm.at[sidx.at[w + 2]], stg_a, sem_a)

        pltpu.sync_copy(stg_b, acc.at[didx.at[w + 1]], add=True)
        return carry

    lax.fori_loop(0, NWIN // 2, pair, 0, unroll=False)
    plsc.subcore_barrier()

    @pl.when(s < NS - 1)
    def _():
        pltpu.sync_copy(acc.at[pl.ds(s * STRIPE, STRIPE)],
                        out.at[c, pl.ds(s * STRIPE, STRIPE)])

    @pl.when(s == NS - 1)
    def _():
        pltpu.sync_copy(acc.at[pl.ds((NS - 1) * STRIPE, TAIL)],
                        out.at[c, pl.ds((NS - 1) * STRIPE, TAIL)])


def _agg_kernel(g, zeros2, srcp, dstp):
    return pl.kernel(
        _agg_body,
        out_type=jax.ShapeDtypeStruct((NC, N, H), jnp.float32),
        mesh=plsc.VectorSubcoreMesh(**_MESH),
        compiler_params=_SC_PARAMS,
        scratch_types=[
            pltpu.VMEM_SHARED((ROWS_ACC, H), jnp.float32),
            pltpu.VMEM((NWIN, WIN), jnp.int32),
            pltpu.VMEM((NWIN, WIN), jnp.int32),
            pltpu.VMEM((WIN, H), jnp.float32),
            pltpu.VMEM((WIN, H), jnp.float32),
            pltpu.SemaphoreType.DMA,
            pltpu.SemaphoreType.DMA,
        ],
    )(g, zeros2, srcp, dstp)


def _tc1_body(degT_ref, x_ref, w1_ref, dis_ref, g1_ref):
    deg = jnp.sum(degT_ref[...], axis=1, keepdims=True) + 1.0
    dis = lax.rsqrt(deg)
    dis_ref[...] = dis
    g1_ref[...] = jnp.dot(x_ref[...], w1_ref[...],
                          preferred_element_type=jnp.float32) * dis


def _tc1(degT, x, W1):
    return pl.pallas_call(
        _tc1_body,
        out_shape=(jax.ShapeDtypeStruct((N, 1), jnp.float32),
                   jax.ShapeDtypeStruct((N, H), jnp.float32)),
    )(degT, x, W1)


def _tcmid_body(p_ref, g_ref, dis_ref, b_ref, w_ref, out_ref):
    dis = dis_ref[...]
    t = p_ref[0] + p_ref[1] + g_ref[...]
    h = jnp.maximum(t * dis + b_ref[...], 0.0)
    out_ref[...] = jnp.dot(h, w_ref[...],
                           preferred_element_type=jnp.float32) * dis


def _tcmid(p, g, dis, b, W):
    return pl.pallas_call(
        _tcmid_body,
        out_shape=jax.ShapeDtypeStruct((N, H), jnp.float32),
    )(p, g, dis, b, W)


def _tc4_body(p_ref, g_ref, dis_ref, b_ref, lw_ref, lb_ref, batch_ref, out_ref):
    t = p_ref[0] + p_ref[1] + g_ref[...]
    h = t * dis_ref[...] + b_ref[...]
    s = jnp.dot(h, lw_ref[...], preferred_element_type=jnp.float32)
    gids = lax.broadcasted_iota(jnp.int32, (N, G), 1)
    mask = (batch_ref[...] == gids).astype(jnp.float32)
    seg = lax.dot_general(mask, s, (((0,), (0,)), ((), ())),
                          preferred_element_type=jnp.float32)
    ones = jnp.ones((N, 1), jnp.float32)
    cnt = lax.dot_general(mask, ones, (((0,), (0,)), ((), ())),
                          preferred_element_type=jnp.float32)
    out_ref[...] = seg / jnp.maximum(cnt, 1.0) + lb_ref[...]


def _tc4(p, g, dis, b, lin_W, lin_b, batch2):
    return pl.pallas_call(
        _tc4_body,
        out_shape=jax.ShapeDtypeStruct((G, 1), jnp.float32),
    )(p, g, dis, b, lin_W, lin_b, batch2)


def kernel(x, edge_index, batch, W1, b1, W2, b2, W3, b3, lin_W, lin_b):
    src = edge_index[0]
    dst = edge_index[1]
    srcp = jnp.concatenate(
        [src, jnp.zeros((EPAD,), jnp.int32)]).reshape(NW, NWIN, WIN)
    dstp = jnp.concatenate(
        [dst, jnp.full((EPAD,), N, jnp.int32)]).reshape(NW, NWIN, WIN)
    zeros1 = jnp.zeros((ROWS_D,), jnp.float32)
    zeros2 = jnp.zeros((ROWS_ACC, H), jnp.float32)

    degp = _deg_kernel(dstp, zeros1).reshape(NC, ROWS_D)
    degT = degp[:, :N].T

    dis, g1 = _tc1(degT, x, W1)
    p1 = _agg_kernel(g1, zeros2, srcp, dstp)
    g2 = _tcmid(p1, g1, dis, b1.reshape(1, H), W2)
    p2 = _agg_kernel(g2, zeros2, srcp, dstp)
    g3 = _tcmid(p2, g2, dis, b2.reshape(1, H), W3)
    p3 = _agg_kernel(g3, zeros2, srcp, dstp)
    return _tc4(p3, g3, dis, b3.reshape(1, H), lin_W,
                lin_b.reshape(1, 1), batch.reshape(N, 1).astype(jnp.int32))


# speedup vs baseline: 2.3759x; 2.3759x over previous
import functools

import jax
import jax.numpy as jnp
from jax import lax
from jax.experimental import pallas as pl
from jax.experimental.pallas import tpu as pltpu
from jax.experimental.pallas import tpu_sc as plsc

N = 10000
E = 320000
D_IN = 128
H = 64
G = 128

NC = 2
NS = 16
NW = NC * NS
WIN = 160
NWIN = 64
EPAD = NW * NWIN * WIN - E
ROWS_ACC = 10112
ROWS_D = 10240

ACC_PER_SUB = ROWS_ACC // NS
D_PER_SUB = ROWS_D // NS
STRIPE = 640
TAIL = N - STRIPE * (NS - 1)

_MESH = dict(core_axis_name="c", subcore_axis_name="s",
             num_cores=NC, num_subcores=NS)
_SC_PARAMS = pltpu.CompilerParams(use_tc_tiling_on_sc=False)


def _deg_body(dstp, zeros1, out, dacc, didx, ones):
    c = lax.axis_index("c")
    s = lax.axis_index("s")
    wid = s * NC + c
    pltpu.sync_copy(zeros1.at[pl.ds(s * D_PER_SUB, D_PER_SUB)],
                    dacc.at[pl.ds(s * D_PER_SUB, D_PER_SUB)])
    pltpu.sync_copy(dstp.at[wid], didx)
    for i in range(WIN // 16):
        ones[pl.ds(i * 16, 16)] = jnp.ones((16,), jnp.float32)
    plsc.subcore_barrier()

    def win(w, carry):
        pltpu.sync_copy(ones, dacc.at[didx.at[w]], add=True)
        return carry

    lax.fori_loop(0, NWIN, win, 0, unroll=False)
    plsc.subcore_barrier()
    pltpu.sync_copy(dacc.at[pl.ds(s * D_PER_SUB, D_PER_SUB)],
                    out.at[pl.ds(c * ROWS_D + s * D_PER_SUB, D_PER_SUB)])


def _deg_kernel(dstp, zeros1):
    return pl.kernel(
        _deg_body,
        out_type=jax.ShapeDtypeStruct((NC * ROWS_D,), jnp.float32),
        mesh=plsc.VectorSubcoreMesh(**_MESH),
        compiler_params=_SC_PARAMS,
        scratch_types=[
            pltpu.VMEM_SHARED((ROWS_D,), jnp.float32),
            pltpu.VMEM((NWIN, WIN), jnp.int32),
            pltpu.VMEM((WIN,), jnp.float32),
        ],
    )(dstp, zeros1)


def _agg_body(g_hbm, zeros2, srcp, dstp, out, g_tbl, acc, sidx, didx,
              stg_a, stg_b, sem_a, sem_b):
    c = lax.axis_index("c")
    s = lax.axis_index("s")
    wid = s * NC + c
    @pl.when(s < NS - 1)
    def _():
        pltpu.sync_copy(g_hbm.at[pl.ds(s * STRIPE, STRIPE)],
                        g_tbl.at[pl.ds(s * STRIPE, STRIPE)])

    @pl.when(s == NS - 1)
    def _():
        pltpu.sync_copy(g_hbm.at[pl.ds((NS - 1) * STRIPE, TAIL)],
                        g_tbl.at[pl.ds((NS - 1) * STRIPE, TAIL)])

    pltpu.sync_copy(zeros2.at[pl.ds(s * ACC_PER_SUB, ACC_PER_SUB)],
                    acc.at[pl.ds(s * ACC_PER_SUB, ACC_PER_SUB)])
    pltpu.sync_copy(srcp.at[wid], sidx)
    pltpu.sync_copy(dstp.at[wid], didx)
    plsc.subcore_barrier()

    pltpu.async_copy(g_tbl.at[sidx.at[0]], stg_a, sem_a)

    def pair(k, carry):
        w = 2 * k
        pltpu.make_async_copy(g_tbl.at[sidx.at[w]], stg_a, sem_a).wait()
        pltpu.async_copy(g_tbl.at[sidx.at[w + 1]], stg_b, sem_b)
        pltpu.sync_copy(stg_a, acc.at[didx.at[w]], add=True)
        pltpu.make_async_copy(g_tbl.at[sidx.at[w + 1]], stg_b, sem_b).wait()

        @pl.when(k < NWIN // 2 - 1)
        def _():
            pltpu.async_copy(g_tbl.at[sidx.at[w + 2]], stg_a, sem_a)

        pltpu.sync_copy(stg_b, acc.at[didx.at[w + 1]], add=True)
        return carry

    lax.fori_loop(0, NWIN // 2, pair, 0, unroll=False)
    plsc.subcore_barrier()

    @pl.when(s < NS - 1)
    def _():
        pltpu.sync_copy(acc.at[pl.ds(s * STRIPE, STRIPE)],
                        out.at[c, pl.ds(s * STRIPE, STRIPE)])

    @pl.when(s == NS - 1)
    def _():
        pltpu.sync_copy(acc.at[pl.ds((NS - 1) * STRIPE, TAIL)],
                        out.at[c, pl.ds((NS - 1) * STRIPE, TAIL)])


def _agg_kernel(g, zeros2, srcp, dstp):
    return pl.kernel(
        _agg_body,
        out_type=jax.ShapeDtypeStruct((NC, N, H), jnp.float32),
        mesh=plsc.VectorSubcoreMesh(**_MESH),
        compiler_params=_SC_PARAMS,
        scratch_types=[
            pltpu.VMEM_SHARED((N, H), jnp.float32),
            pltpu.VMEM_SHARED((ROWS_ACC, H), jnp.float32),
            pltpu.VMEM((NWIN, WIN), jnp.int32),
            pltpu.VMEM((NWIN, WIN), jnp.int32),
            pltpu.VMEM((WIN, H), jnp.float32),
            pltpu.VMEM((WIN, H), jnp.float32),
            pltpu.SemaphoreType.DMA,
            pltpu.SemaphoreType.DMA,
        ],
    )(g, zeros2, srcp, dstp)


def _tc1_body(degT_ref, x_ref, w1_ref, dis_ref, g1_ref):
    deg = jnp.sum(degT_ref[...], axis=1, keepdims=True) + 1.0
    dis = lax.rsqrt(deg)
    dis_ref[...] = dis
    g1_ref[...] = jnp.dot(x_ref[...], w1_ref[...],
                          preferred_element_type=jnp.float32) * dis


def _tc1(degT, x, W1):
    return pl.pallas_call(
        _tc1_body,
        out_shape=(jax.ShapeDtypeStruct((N, 1), jnp.float32),
                   jax.ShapeDtypeStruct((N, H), jnp.float32)),
    )(degT, x, W1)


def _tcmid_body(p_ref, g_ref, dis_ref, b_ref, w_ref, out_ref):
    dis = dis_ref[...]
    t = p_ref[0] + p_ref[1] + g_ref[...]
    h = jnp.maximum(t * dis + b_ref[...], 0.0)
    out_ref[...] = jnp.dot(h, w_ref[...],
                           preferred_element_type=jnp.float32) * dis


def _tcmid(p, g, dis, b, W):
    return pl.pallas_call(
        _tcmid_body,
        out_shape=jax.ShapeDtypeStruct((N, H), jnp.float32),
    )(p, g, dis, b, W)


def _tc4_body(p_ref, g_ref, dis_ref, b_ref, lw_ref, lb_ref, batch_ref, out_ref):
    t = p_ref[0] + p_ref[1] + g_ref[...]
    h = t * dis_ref[...] + b_ref[...]
    s = jnp.dot(h, lw_ref[...], preferred_element_type=jnp.float32)
    gids = lax.broadcasted_iota(jnp.int32, (N, G), 1)
    mask = (batch_ref[...] == gids).astype(jnp.float32)
    seg = lax.dot_general(mask, s, (((0,), (0,)), ((), ())),
                          preferred_element_type=jnp.float32)
    ones = jnp.ones((N, 1), jnp.float32)
    cnt = lax.dot_general(mask, ones, (((0,), (0,)), ((), ())),
                          preferred_element_type=jnp.float32)
    out_ref[...] = seg / jnp.maximum(cnt, 1.0) + lb_ref[...]


def _tc4(p, g, dis, b, lin_W, lin_b, batch2):
    return pl.pallas_call(
        _tc4_body,
        out_shape=jax.ShapeDtypeStruct((G, 1), jnp.float32),
    )(p, g, dis, b, lin_W, lin_b, batch2)


def kernel(x, edge_index, batch, W1, b1, W2, b2, W3, b3, lin_W, lin_b):
    src = edge_index[0]
    dst = edge_index[1]
    srcp = jnp.concatenate(
        [src, jnp.zeros((EPAD,), jnp.int32)]).reshape(NW, NWIN, WIN)
    dstp = jnp.concatenate(
        [dst, jnp.full((EPAD,), N, jnp.int32)]).reshape(NW, NWIN, WIN)
    zeros1 = jnp.zeros((ROWS_D,), jnp.float32)
    zeros2 = jnp.zeros((ROWS_ACC, H), jnp.float32)

    degp = _deg_kernel(dstp, zeros1).reshape(NC, ROWS_D)
    degT = degp[:, :N].T

    dis, g1 = _tc1(degT, x, W1)
    p1 = _agg_kernel(g1, zeros2, srcp, dstp)
    g2 = _tcmid(p1, g1, dis, b1.reshape(1, H), W2)
    p2 = _agg_kernel(g2, zeros2, srcp, dstp)
    g3 = _tcmid(p2, g2, dis, b2.reshape(1, H), W3)
    p3 = _agg_kernel(g3, zeros2, srcp, dstp)
    return _tc4(p3, g3, dis, b3.reshape(1, H), lin_W,
                lin_b.reshape(1, 1), batch.reshape(N, 1).astype(jnp.int32))
